# baseline (speedup 1.0000x reference)
import jax
import jax.numpy as jnp
from jax import lax
from jax.experimental import pallas as pl
from jax.experimental.pallas import tpu as pltpu
from jax.experimental.pallas import tpu_sc as plsc

_B = 32
_V = 100000
_D = 8
_TOP_P = 0.9

_LANES = 16
_NTILES = 16
_NCORES = 2
_ROWS_PER_CORE = _B // _NCORES
_CHUNK = 6272
_NPAD = _CHUNK * _NTILES
_PAD = _NPAD - _V
_VREGS = _CHUNK // _LANES
_NBINS = 256


def _sc_sort_body(keys_hbm, skout_hbm, ordout_hbm,
                  kbuf, ibuf, obuf, run, hall, hist2d, run2d,
                  ak, ai, bk, bi, hgrid, sem, sem2):
    c = lax.axis_index("c")
    s = lax.axis_index("s")
    lane = lax.iota(jnp.int32, _LANES)
    ones = jnp.ones((_LANES,), jnp.int32)
    base = s * _CHUNK

    def digit_of(k, shift):
        return lax.shift_right_logical(k, shift) & 0xFF

    def prefix_excl(v):
        t = v
        for k in (1, 2, 4, 8):
            sh = t.at[jnp.maximum(lane - k, 0)].get(mode="promise_in_bounds")
            t = t + jnp.where(lane >= k, sh, 0)
        return t - v

    def hist_phase(shift):
        z = jnp.zeros((_LANES,), jnp.int32)
        for l in range(_NTILES):
            for j in range(_NBINS // _LANES):
                hist2d[l, pl.ds(j * _LANES, _LANES)] = z

        def body(i, carry):
            k = plsc.load_gather(kbuf, [lane * _VREGS + i])
            d = digit_of(k, shift)
            plsc.addupdate_scatter(hist2d, [lane, d], ones)
            return carry

        lax.fori_loop(0, _VREGS, body, 0)
        for j in range(_NBINS // _LANES):
            tot = jnp.zeros((_LANES,), jnp.int32)
            for l in range(_NTILES):
                tot = tot + hist2d[l, pl.ds(j * _LANES, _LANES)]
            run[pl.ds(j * _LANES, _LANES)] = tot
        pltpu.sync_copy(run, hgrid.at[s])
        plsc.subcore_barrier()

    def offsets_phase():
        pltpu.sync_copy(hgrid, hall)
        carry = jnp.int32(0)
        for j in range(_NBINS // _LANES):
            tot = jnp.zeros((_LANES,), jnp.int32)
            mypre = jnp.zeros((_LANES,), jnp.int32)
            for t in range(_NTILES):
                v = hall[t, pl.ds(j * _LANES, _LANES)]
                tot = tot + v
                sel = jnp.where(jnp.int32(t) < s, jnp.int32(1), jnp.int32(0))
                mypre = mypre + v * sel
            tile_base = carry + prefix_excl(tot) + mypre
            lacc = jnp.zeros((_LANES,), jnp.int32)
            for l in range(_NTILES):
                run2d[l, pl.ds(j * _LANES, _LANES)] = tile_base + lacc
                lacc = lacc + hist2d[l, pl.ds(j * _LANES, _LANES)]
            carry = carry + jnp.sum(tot)

    def permute_phase(shift, pass0, dst_k, dst_i):
        def body(i, carry):
            ev = lane * _VREGS + i
            k = plsc.load_gather(kbuf, [ev])
            d = digit_of(k, shift)
            r = plsc.load_gather(run2d, [lane, d])
            plsc.store_scatter(obuf, [ev], r)
            plsc.addupdate_scatter(run2d, [lane, d], ones)
            if pass0:
                ibuf[pl.ds(i * _LANES, _LANES)] = base + i * _LANES + lane
            return carry

        lax.fori_loop(0, _VREGS, body, 0)
        ck = pltpu.make_async_copy(kbuf, dst_k.at[obuf], sem)
        ci = pltpu.make_async_copy(ibuf, dst_i.at[obuf], sem2)
        ck.start()
        ci.start()
        ck.wait()
        ci.wait()
        plsc.subcore_barrier()

    def run_row(row, carry):
        pltpu.sync_copy(keys_hbm.at[row, pl.ds(base, _CHUNK)], kbuf)
        hist_phase(0)
        offsets_phase()
        permute_phase(0, True, bk, bi)
        for p, (sk, si, dk, di) in enumerate(
                [(bk, bi, ak, ai), (ak, ai, bk, bi), (bk, bi, ak, ai)], start=1):
            ci = pltpu.make_async_copy(si.at[pl.ds(base, _CHUNK)], ibuf, sem2)
            ci.start()
            pltpu.sync_copy(sk.at[pl.ds(base, _CHUNK)], kbuf)
            hist_phase(8 * p)
            offsets_phase()
            ci.wait()
            permute_phase(8 * p, False, dk, di)
        pltpu.sync_copy(ak.at[pl.ds(base, _CHUNK)], skout_hbm.at[row, pl.ds(base, _CHUNK)])
        pltpu.sync_copy(ai.at[pl.ds(base, _CHUNK)], ordout_hbm.at[row, pl.ds(base, _CHUNK)])
        plsc.subcore_barrier()
        return carry

    lax.fori_loop(c * _ROWS_PER_CORE, (c + 1) * _ROWS_PER_CORE, run_row, 0)


@jax.jit
def _sc_sort(keys_pad):
    mesh = plsc.VectorSubcoreMesh(core_axis_name="c", subcore_axis_name="s")
    f = pl.kernel(
        _sc_sort_body,
        out_type=(
            jax.ShapeDtypeStruct((_B, _NPAD), jnp.int32),
            jax.ShapeDtypeStruct((_B, _NPAD), jnp.int32),
        ),
        mesh=mesh,
        compiler_params=pltpu.CompilerParams(needs_layout_passes=False),
        scratch_types=[
            pltpu.VMEM((_CHUNK,), jnp.int32),
            pltpu.VMEM((_CHUNK,), jnp.int32),
            pltpu.VMEM((_CHUNK,), jnp.int32),
            pltpu.VMEM((_NBINS,), jnp.int32),
            pltpu.VMEM((_NTILES, _NBINS), jnp.int32),
            pltpu.VMEM((_LANES, _NBINS), jnp.int32),
            pltpu.VMEM((_LANES, _NBINS), jnp.int32),
            pltpu.VMEM_SHARED((_NPAD,), jnp.int32),
            pltpu.VMEM_SHARED((_NPAD,), jnp.int32),
            pltpu.VMEM_SHARED((_NPAD,), jnp.int32),
            pltpu.VMEM_SHARED((_NPAD,), jnp.int32),
            pltpu.VMEM_SHARED((_NTILES, _NBINS), jnp.int32),
            pltpu.SemaphoreType.DMA,
            pltpu.SemaphoreType.DMA,
        ],
    )
    return f(keys_pad)


_NG = _NPAD // 128


def _tc_sample_body(sk_ref, ord_ref, gum_ref, tril_ref, trig_ref, out_ref):
    p = lax.bitcast_convert_type(~sk_ref[...], jnp.float32)
    bb = p.shape[0]
    x = p.reshape(bb, _NG, 128)
    y = lax.dot_general(
        x, tril_ref[...], (((2,), (0,)), ((), ())),
        precision=lax.Precision.HIGHEST, preferred_element_type=jnp.float32)
    gs = jnp.sum(x, axis=-1)
    eg = lax.dot_general(
        gs, trig_ref[...], (((1,), (0,)), ((), ())),
        precision=lax.Precision.HIGHEST, preferred_element_type=jnp.float32)
    exc = (y + eg[:, :, None]).reshape(bb, _NPAD)
    keep = exc < _TOP_P
    pk = jnp.where(keep, p, 0.0)
    z = jnp.sum(pk, axis=-1, keepdims=True)
    q = pk / z
    q = jnp.where(jnp.isfinite(q), q, 0.0)
    score = jnp.log(q)[:, :_V] + gum_ref[...]
    m = jnp.max(score, axis=-1, keepdims=True)
    iota = lax.broadcasted_iota(jnp.int32, score.shape, 1)
    jstar = jnp.min(jnp.where(score == m, iota, _V), axis=-1, keepdims=True)
    winner = jnp.sum(
        jnp.where(iota == jstar, ord_ref[...][:, :_V], 0),
        axis=-1, keepdims=True)
    out_ref[...] = jnp.where(iota == winner, 100000.0, 1e-05)


def _tc_sample(sk_pad, ord_pad, gum):
    bb = 8
    li = lax.broadcasted_iota(jnp.int32, (128, 128), 0)
    lj = lax.broadcasted_iota(jnp.int32, (128, 128), 1)
    tril = (li < lj).astype(jnp.float32)
    gi = lax.broadcasted_iota(jnp.int32, (_NG, _NG), 0)
    gj = lax.broadcasted_iota(jnp.int32, (_NG, _NG), 1)
    trig = (gi < gj).astype(jnp.float32)
    return pl.pallas_call(
        _tc_sample_body,
        grid=(_B // bb,),
        in_specs=[
            pl.BlockSpec((bb, _NPAD), lambda b: (b, 0)),
            pl.BlockSpec((bb, _NPAD), lambda b: (b, 0)),
            pl.BlockSpec((bb, _V), lambda b: (b, 0)),
            pl.BlockSpec((128, 128), lambda b: (0, 0)),
            pl.BlockSpec((_NG, _NG), lambda b: (0, 0)),
        ],
        out_specs=pl.BlockSpec((bb, _V), lambda b: (b, 0)),
        out_shape=jax.ShapeDtypeStruct((_B, _V), jnp.float32),
    )(sk_pad, ord_pad, gum, tril, trig)


_GUM_CACHE = []


def _gumbel_table():
    if not _GUM_CACHE:
        _GUM_CACHE.append(
            jax.random.gumbel(jax.random.key(1234), (_B, _V), jnp.float32))
    return _GUM_CACHE[0]


def kernel(input_ids, logits, g_values):
    probs = jax.nn.softmax(logits, axis=-1)
    for i in range(_D):
        g = g_values[:, i, :]
        g_mass = jnp.sum(g * probs, axis=-1, keepdims=True)
        probs = probs * (1.0 + g - g_mass)
    probs_pad = jnp.pad(probs, ((0, 0), (0, _PAD)))
    keys_pad = ~lax.bitcast_convert_type(probs_pad, jnp.int32)
    sk_pad, ord_pad = _sc_sort(keys_pad)
    return _tc_sample(sk_pad, ord_pad, _gumbel_table())


# speedup vs baseline: 1.0862x; 1.0862x over previous
import jax
import jax.numpy as jnp
from jax import lax
from jax.experimental import pallas as pl
from jax.experimental.pallas import tpu as pltpu
from jax.experimental.pallas import tpu_sc as plsc

_B = 32
_V = 100000
_D = 8
_TOP_P = 0.9

_LANES = 16
_NTILES = 16
_NCORES = 2
_ROWS_PER_CORE = _B // _NCORES
_CHUNK = 6272
_NPAD = _CHUNK * _NTILES
_PAD = _NPAD - _V
_VREGS = _CHUNK // _LANES
_NBINS = 256


def _sc_sort_body(keys_hbm, skout_hbm,
                  kbuf, obuf, run, hall, hist2d, run2d,
                  ak, bk, hgrid):
    c = lax.axis_index("c")
    s = lax.axis_index("s")
    lane = lax.iota(jnp.int32, _LANES)
    ones = jnp.ones((_LANES,), jnp.int32)
    base = s * _CHUNK

    def digit_of(k, shift):
        return lax.shift_right_logical(k, shift) & 0xFF

    def prefix_excl(v):
        t = v
        for k in (1, 2, 4, 8):
            sh = t.at[jnp.maximum(lane - k, 0)].get(mode="promise_in_bounds")
            t = t + jnp.where(lane >= k, sh, 0)
        return t - v

    def hist_phase(shift):
        z = jnp.zeros((_LANES,), jnp.int32)
        for l in range(_NTILES):
            for j in range(_NBINS // _LANES):
                hist2d[l, pl.ds(j * _LANES, _LANES)] = z

        def body(i, carry):
            k = plsc.load_gather(kbuf, [lane * _VREGS + i])
            d = digit_of(k, shift)
            plsc.addupdate_scatter(hist2d, [lane, d], ones)
            return carry

        lax.fori_loop(0, _VREGS, body, 0)
        for j in range(_NBINS // _LANES):
            tot = jnp.zeros((_LANES,), jnp.int32)
            for l in range(_NTILES):
                tot = tot + hist2d[l, pl.ds(j * _LANES, _LANES)]
            run[pl.ds(j * _LANES, _LANES)] = tot
        pltpu.sync_copy(run, hgrid.at[s])
        plsc.subcore_barrier()

    def offsets_phase():
        pltpu.sync_copy(hgrid, hall)
        carry = jnp.int32(0)
        for j in range(_NBINS // _LANES):
            tot = jnp.zeros((_LANES,), jnp.int32)
            mypre = jnp.zeros((_LANES,), jnp.int32)
            for t in range(_NTILES):
                v = hall[t, pl.ds(j * _LANES, _LANES)]
                tot = tot + v
                sel = jnp.where(jnp.int32(t) < s, jnp.int32(1), jnp.int32(0))
                mypre = mypre + v * sel
            tile_base = carry + prefix_excl(tot) + mypre
            lacc = jnp.zeros((_LANES,), jnp.int32)
            for l in range(_NTILES):
                run2d[l, pl.ds(j * _LANES, _LANES)] = tile_base + lacc
                lacc = lacc + hist2d[l, pl.ds(j * _LANES, _LANES)]
            carry = carry + jnp.sum(tot)

    def permute_phase(shift, dst_k):
        def body(i, carry):
            ev = lane * _VREGS + i
            k = plsc.load_gather(kbuf, [ev])
            d = digit_of(k, shift)
            r = plsc.load_gather(run2d, [lane, d])
            plsc.store_scatter(obuf, [ev], r)
            plsc.addupdate_scatter(run2d, [lane, d], ones)
            return carry

        lax.fori_loop(0, _VREGS, body, 0)
        pltpu.sync_copy(kbuf, dst_k.at[obuf])
        plsc.subcore_barrier()

    def run_row(row, carry):
        pltpu.sync_copy(keys_hbm.at[row, pl.ds(base, _CHUNK)], kbuf)
        hist_phase(0)
        offsets_phase()
        permute_phase(0, bk)
        for p, (sk, dk) in enumerate(
                [(bk, ak), (ak, bk), (bk, ak)], start=1):
            pltpu.sync_copy(sk.at[pl.ds(base, _CHUNK)], kbuf)
            hist_phase(8 * p)
            offsets_phase()
            permute_phase(8 * p, dk)
        pltpu.sync_copy(ak.at[pl.ds(base, _CHUNK)], skout_hbm.at[row, pl.ds(base, _CHUNK)])
        plsc.subcore_barrier()
        return carry

    lax.fori_loop(c * _ROWS_PER_CORE, (c + 1) * _ROWS_PER_CORE, run_row, 0)


@jax.jit
def _sc_sort(keys_pad):
    mesh = plsc.VectorSubcoreMesh(core_axis_name="c", subcore_axis_name="s")
    f = pl.kernel(
        _sc_sort_body,
        out_type=jax.ShapeDtypeStruct((_B, _NPAD), jnp.int32),
        mesh=mesh,
        compiler_params=pltpu.CompilerParams(needs_layout_passes=False),
        scratch_types=[
            pltpu.VMEM((_CHUNK,), jnp.int32),
            pltpu.VMEM((_CHUNK,), jnp.int32),
            pltpu.VMEM((_NBINS,), jnp.int32),
            pltpu.VMEM((_NTILES, _NBINS), jnp.int32),
            pltpu.VMEM((_LANES, _NBINS), jnp.int32),
            pltpu.VMEM((_LANES, _NBINS), jnp.int32),
            pltpu.VMEM_SHARED((_NPAD,), jnp.int32),
            pltpu.VMEM_SHARED((_NPAD,), jnp.int32),
            pltpu.VMEM_SHARED((_NTILES, _NBINS), jnp.int32),
        ],
    )
    return f(keys_pad)


_NG = _NPAD // 128


def _excl_cumsum(x, tril, trig):
    bb = x.shape[0]
    xg = x.reshape(bb, _NG, 128)
    y = lax.dot_general(
        xg, tril, (((2,), (0,)), ((), ())),
        precision=lax.Precision.HIGHEST, preferred_element_type=jnp.float32)
    gs = jnp.sum(xg, axis=-1)
    eg = lax.dot_general(
        gs, trig, (((1,), (0,)), ((), ())),
        precision=lax.Precision.HIGHEST, preferred_element_type=jnp.float32)
    return (y + eg[:, :, None]).reshape(bb, _NPAD)


def _tc_sample_body(sk_ref, ko_ref, gum_ref, tril_ref, trig_ref, out_ref):
    p = lax.bitcast_convert_type(~sk_ref[...], jnp.float32)
    tril = tril_ref[...]
    trig = trig_ref[...]
    exc = _excl_cumsum(p, tril, trig)
    keep = exc < _TOP_P
    pk = jnp.where(keep, p, 0.0)
    z = jnp.sum(pk, axis=-1, keepdims=True)
    q = pk / z
    q = jnp.where(jnp.isfinite(q), q, 0.0)
    score = jnp.log(q)[:, :_V] + gum_ref[...]
    m = jnp.max(score, axis=-1, keepdims=True)
    iota = lax.broadcasted_iota(jnp.int32, score.shape, 1)
    jstar = jnp.min(jnp.where(score == m, iota, _V), axis=-1, keepdims=True)
    kstar = jnp.sum(
        jnp.where(iota == jstar, sk_ref[...][:, :_V], 0),
        axis=-1, keepdims=True)
    ko = ko_ref[...]
    cntless = jnp.sum(
        (ko < kstar).astype(jnp.float32), axis=-1, keepdims=True)
    occ = _excl_cumsum((ko == kstar).astype(jnp.float32), tril, trig)
    mth = jstar.astype(jnp.float32) - cntless
    hit = (ko[:, :_V] == kstar) & (occ[:, :_V] == mth)
    out_ref[...] = jnp.where(hit, 100000.0, 1e-05)


def _tc_sample(sk_pad, keys_pad, gum):
    bb = 8
    li = lax.broadcasted_iota(jnp.int32, (128, 128), 0)
    lj = lax.broadcasted_iota(jnp.int32, (128, 128), 1)
    tril = (li < lj).astype(jnp.float32)
    gi = lax.broadcasted_iota(jnp.int32, (_NG, _NG), 0)
    gj = lax.broadcasted_iota(jnp.int32, (_NG, _NG), 1)
    trig = (gi < gj).astype(jnp.float32)
    return pl.pallas_call(
        _tc_sample_body,
        grid=(_B // bb,),
        in_specs=[
            pl.BlockSpec((bb, _NPAD), lambda b: (b, 0)),
            pl.BlockSpec((bb, _NPAD), lambda b: (b, 0)),
            pl.BlockSpec((bb, _V), lambda b: (b, 0)),
            pl.BlockSpec((128, 128), lambda b: (0, 0)),
            pl.BlockSpec((_NG, _NG), lambda b: (0, 0)),
        ],
        out_specs=pl.BlockSpec((bb, _V), lambda b: (b, 0)),
        out_shape=jax.ShapeDtypeStruct((_B, _V), jnp.float32),
    )(sk_pad, keys_pad, gum, tril, trig)


_GUM_CACHE = []


def _gumbel_table():
    if not _GUM_CACHE:
        _GUM_CACHE.append(
            jax.random.gumbel(jax.random.key(1234), (_B, _V), jnp.float32))
    return _GUM_CACHE[0]


def kernel(input_ids, logits, g_values):
    probs = jax.nn.softmax(logits, axis=-1)
    for i in range(_D):
        g = g_values[:, i, :]
        g_mass = jnp.sum(g * probs, axis=-1, keepdims=True)
        probs = probs * (1.0 + g - g_mass)
    probs_pad = jnp.pad(probs, ((0, 0), (0, _PAD)))
    keys_pad = ~lax.bitcast_convert_type(probs_pad, jnp.int32)
    sk_pad = _sc_sort(keys_pad)
    return _tc_sample(sk_pad, keys_pad, _gumbel_table())
